# Initial kernel scaffold; baseline (speedup 1.0000x reference)
import jax
import jax.numpy as jnp
from jax.experimental import pallas as pl


def kernel(x, table, offsets, bias):
    raise NotImplementedError("write your pallas kernel here")


# speedup vs baseline: 90.2801x; 90.2801x over previous
import functools

import jax
import jax.numpy as jnp
from jax import lax
from jax.experimental import pallas as pl
from jax.experimental.pallas import tpu as pltpu
from jax.experimental.pallas import tpu_sc as plsc


def _build(B, F, V):
    info = plsc.get_sparse_core_info()
    NC, NS, L = info.num_cores, info.num_subcores, info.num_lanes
    NW = NC * NS
    bpw = B // NW
    groups = bpw // L
    FP = 32

    mesh = plsc.VectorSubcoreMesh(core_axis_name="c", subcore_axis_name="s")

    @functools.partial(
        pl.kernel,
        out_type=jax.ShapeDtypeStruct((B,), jnp.float32),
        mesh=mesh,
        compiler_params=pltpu.CompilerParams(needs_layout_passes=False),
        scratch_types=[
            pltpu.VMEM((V,), jnp.float32),
            pltpu.VMEM((F, bpw), jnp.int32),
            pltpu.VMEM((bpw,), jnp.float32),
            pltpu.VMEM((FP,), jnp.int32),
            pltpu.VMEM((16,), jnp.float32),
        ],
    )
    def k(x_hbm, tab_hbm, off_hbm, bias_hbm, out_hbm, tab_v, x_v, o_v, off_v, b_v):
        wid = lax.axis_index("s") * NC + lax.axis_index("c")
        pltpu.sync_copy(tab_hbm, tab_v)
        pltpu.sync_copy(off_hbm, off_v)
        pltpu.sync_copy(bias_hbm, b_v)
        pltpu.sync_copy(x_hbm.at[wid], x_v)

        bias_vec = b_v[...]
        off_vecs = [
            plsc.load_gather(off_v, [jnp.full((L,), f + 1, jnp.int32)])
            for f in range(F)
        ]

        for g in range(groups):
            col = g * L
            acc = bias_vec
            for f in range(F):
                idx = x_v[f, pl.ds(col, L)] + off_vecs[f]
                acc = acc + plsc.load_gather(tab_v, [idx])
            o_v[pl.ds(col, L)] = acc
        pltpu.sync_copy(o_v, out_hbm.at[pl.ds(wid * bpw, bpw)])

    return k


def kernel(x, table, offsets, bias):
    B, F = x.shape
    V = table.shape[0]
    NW = 32
    bpw = B // NW
    x_blocks = (
        x.astype(jnp.int32).T.reshape(F, NW, bpw).transpose(1, 0, 2)
    )
    off_pad = jnp.zeros((32,), jnp.int32).at[1:F + 1].set(offsets.astype(jnp.int32))
    bias_pad = jnp.broadcast_to(bias.astype(jnp.float32), (16,))
    out = _build(B, F, V)(x_blocks, table.reshape(-1), off_pad, bias_pad)
    return out[:, None]
